# Initial kernel scaffold; baseline (speedup 1.0000x reference)
import functools

import jax
import jax.numpy as jnp
from jax import lax
from jax.experimental import pallas as pl
from jax.experimental.pallas import tpu as pltpu
from jax.experimental.pallas import tpu_sc as plsc

DIM = 32
NNB = 16
NRELP = 48
NC, NS, L = 2, 16, 16
NW = NC * NS


def _mesh():
    return plsc.VectorSubcoreMesh(core_axis_name="c", subcore_axis_name="s")


def _user_gather(u, usr_table):
    B = u.shape[0]
    bpw = B // NW

    @functools.partial(
        pl.kernel,
        out_type=jax.ShapeDtypeStruct((B, DIM), jnp.float32),
        mesh=_mesh(),
        scratch_types=[
            pltpu.VMEM((bpw,), jnp.int32),
            pltpu.VMEM((bpw, DIM), jnp.float32),
        ],
    )
    def k(u_hbm, tab_hbm, out_hbm, idx_v, rows_v):
        wid = lax.axis_index("s") * NC + lax.axis_index("c")
        base = wid * bpw
        pltpu.sync_copy(u_hbm.at[pl.ds(base, bpw)], idx_v)
        pltpu.sync_copy(tab_hbm.at[idx_v], rows_v)
        pltpu.sync_copy(rows_v, out_hbm.at[pl.ds(base, bpw)])

    return k(u, usr_table)


def _exp_scores(user_emb, relT_pad):
    B = user_emb.shape[0]

    def body(ue_ref, rt_ref, out_ref):
        out_ref[...] = jnp.exp(
            jnp.dot(ue_ref[...], rt_ref[...], preferred_element_type=jnp.float32)
        )

    return pl.pallas_call(
        body,
        out_shape=jax.ShapeDtypeStruct((B, NRELP), jnp.float32),
    )(user_emb, relT_pad)


def _gather_aggregate(v, adj, rel, ent_table, E):
    B = v.shape[0]
    bpw = B // NW
    kidx = [jnp.full((L,), k, jnp.int32) for k in range(NNB)]

    @functools.partial(
        pl.kernel,
        out_type=(
            jax.ShapeDtypeStruct((B, DIM), jnp.float32),
            jax.ShapeDtypeStruct((B, NNB, DIM), jnp.float32),
            jax.ShapeDtypeStruct((B, NNB), jnp.float32),
        ),
        mesh=_mesh(),
        scratch_types=[
            pltpu.VMEM((bpw,), jnp.int32),
            pltpu.VMEM((bpw, NNB), jnp.int32),
            pltpu.VMEM((bpw, NNB), jnp.int32),
            pltpu.VMEM((bpw, DIM), jnp.float32),
            pltpu.VMEM((bpw, NRELP), jnp.float32),
            pltpu.VMEM((bpw, NNB, NNB), jnp.int32),
            pltpu.VMEM((bpw, NNB, NNB), jnp.int32),
            pltpu.VMEM((NNB, NNB, DIM), jnp.float32),
            pltpu.VMEM((NNB, DIM), jnp.float32),
            pltpu.VMEM((NNB, DIM), jnp.float32),
            pltpu.VMEM((bpw, DIM), jnp.float32),
            pltpu.VMEM((bpw, NNB), jnp.float32),
            pltpu.VMEM((L,), jnp.float32),
        ],
    )
    def k(v_hbm, adj_hbm, rel_hbm, ent_hbm, e_hbm,
          sum0_hbm, sum1_hbm, w0_hbm,
          VL, E1, Q0, SV0, EC, E2, Q1, Xb, SV1b, SUM1b, SUM0, W0B, wbuf):
        wid = lax.axis_index("s") * NC + lax.axis_index("c")
        base = wid * bpw

        pltpu.sync_copy(v_hbm.at[pl.ds(base, bpw)], VL)
        pltpu.sync_copy(adj_hbm.at[VL], E1)
        pltpu.sync_copy(rel_hbm.at[VL], Q0)
        pltpu.sync_copy(ent_hbm.at[VL], SV0)
        pltpu.sync_copy(e_hbm.at[pl.ds(base, bpw)], EC)
        pltpu.sync_copy(adj_hbm.at[E1], E2)
        pltpu.sync_copy(rel_hbm.at[E1], Q1)

        def seg_weights(b_vec, q):
            e = plsc.load_gather(EC, [b_vec, q])
            s = jnp.sum(e)
            wbuf[...] = e
            return 1.0 / s

        def body(b, carry):
            b_vec = jnp.zeros((L,), jnp.int32) + b
            pltpu.sync_copy(ent_hbm.at[E2.at[b]], Xb)
            pltpu.sync_copy(ent_hbm.at[E1.at[b]], SV1b)
            for p in range(NNB):
                rs = seg_weights(b_vec, Q1[b, p, :])
                acc0 = jnp.zeros((L,), jnp.float32)
                acc1 = jnp.zeros((L,), jnp.float32)
                for kk in range(NNB):
                    bk = plsc.load_gather(wbuf, [kidx[kk]])
                    acc0 = acc0 + bk * Xb[p, kk, 0:L]
                    acc1 = acc1 + bk * Xb[p, kk, L:DIM]
                SUM1b[p, 0:L] = acc0 * rs + SV1b[p, 0:L]
                SUM1b[p, L:DIM] = acc1 * rs + SV1b[p, L:DIM]
            pltpu.sync_copy(SUM1b, sum1_hbm.at[base + b])
            rs0 = seg_weights(b_vec, Q0[b, :])
            a0 = jnp.zeros((L,), jnp.float32)
            a1 = jnp.zeros((L,), jnp.float32)
            for kk in range(NNB):
                bk = plsc.load_gather(wbuf, [kidx[kk]])
                a0 = a0 + bk * SV1b[kk, 0:L]
                a1 = a1 + bk * SV1b[kk, L:DIM]
            W0B[b, :] = wbuf[...] * rs0
            SUM0[b, 0:L] = a0 * rs0 + SV0[b, 0:L]
            SUM0[b, L:DIM] = a1 * rs0 + SV0[b, L:DIM]
            return carry

        lax.fori_loop(0, bpw, body, 0)
        pltpu.sync_copy(SUM0, sum0_hbm.at[pl.ds(base, bpw)])
        pltpu.sync_copy(W0B, w0_hbm.at[pl.ds(base, bpw)])

    return k(v, adj, rel, ent_table, E)


def _dense_finish(user_emb, sum0, sum1_2d, w0, W0T, b0, W1T, b1):
    B = user_emb.shape[0]
    BB = 512
    grid = B // BB

    def body(ue_ref, s0_ref, s1_ref, w0_ref, w0t_ref, b0_ref, w1t_ref, b1_ref,
             out_ref):
        w0t = w0t_ref[...]
        b0v = b0_ref[...]
        w0w = w0_ref[...]
        aggtop = jnp.zeros((BB, DIM), jnp.float32)
        for kk in range(NNB):
            h1k = jax.nn.sigmoid(
                jnp.dot(s1_ref[:, kk * DIM:(kk + 1) * DIM], w0t,
                        preferred_element_type=jnp.float32) + b0v
            )
            aggtop = aggtop + w0w[:, kk:kk + 1] * h1k
        h0 = jax.nn.sigmoid(
            jnp.dot(s0_ref[...], w0t, preferred_element_type=jnp.float32) + b0v
        )
        item = jnp.tanh(
            jnp.dot(h0 + aggtop, w1t_ref[...], preferred_element_type=jnp.float32)
            + b1_ref[...]
        )
        out_ref[...] = jax.nn.sigmoid(jnp.sum(ue_ref[...] * item, axis=1))

    return pl.pallas_call(
        body,
        grid=(grid,),
        in_specs=[
            pl.BlockSpec((BB, DIM), lambda i: (i, 0)),
            pl.BlockSpec((BB, DIM), lambda i: (i, 0)),
            pl.BlockSpec((BB, NNB * DIM), lambda i: (i, 0)),
            pl.BlockSpec((BB, NNB), lambda i: (i, 0)),
            pl.BlockSpec((DIM, DIM), lambda i: (0, 0)),
            pl.BlockSpec((1, DIM), lambda i: (0, 0)),
            pl.BlockSpec((DIM, DIM), lambda i: (0, 0)),
            pl.BlockSpec((1, DIM), lambda i: (0, 0)),
        ],
        out_specs=pl.BlockSpec((BB,), lambda i: (i,)),
        out_shape=jax.ShapeDtypeStruct((B,), jnp.float32),
    )(user_emb, sum0, sum1_2d, w0, W0T, b0, W1T, b1)


def kernel(u, v, adj, rel, train_mode, usr_table, ent_table, rel_table,
           agg_W0, agg_b0, agg_W1, agg_b1):
    del train_mode
    B = v.shape[0]
    u = u.astype(jnp.int32)
    v = v.astype(jnp.int32)
    adj = adj.astype(jnp.int32)
    rel = rel.astype(jnp.int32)

    user_emb = _user_gather(u, usr_table)

    relT_pad = jnp.zeros((DIM, NRELP), jnp.float32).at[:, :rel_table.shape[0]].set(
        rel_table.T)
    E = _exp_scores(user_emb, relT_pad)

    sum0, sum1, w0 = _gather_aggregate(v, adj, rel, ent_table, E)

    return _dense_finish(
        user_emb, sum0, sum1.reshape(B, NNB * DIM), w0,
        agg_W0.T, agg_b0.reshape(1, DIM), agg_W1.T, agg_b1.reshape(1, DIM))


# speedup vs baseline: 15.3954x; 15.3954x over previous
import functools

import jax
import jax.numpy as jnp
from jax import lax
from jax.experimental import pallas as pl
from jax.experimental.pallas import tpu as pltpu
from jax.experimental.pallas import tpu_sc as plsc

DIM = 32
NNB = 16
NRELP = 48
NC, NS, L = 2, 16, 16
NW = NC * NS


def _mesh():
    return plsc.VectorSubcoreMesh(core_axis_name="c", subcore_axis_name="s")


def _user_gather(u, usr_table):
    B = u.shape[0]
    bpw = B // NW

    @functools.partial(
        pl.kernel,
        out_type=jax.ShapeDtypeStruct((B, DIM), jnp.float32),
        mesh=_mesh(),
        scratch_types=[
            pltpu.VMEM((bpw,), jnp.int32),
            pltpu.VMEM((bpw, DIM), jnp.float32),
        ],
        compiler_params=pltpu.CompilerParams(use_tc_tiling_on_sc=False),
    )
    def k(u_hbm, tab_hbm, out_hbm, idx_v, rows_v):
        wid = lax.axis_index("s") * NC + lax.axis_index("c")
        base = wid * bpw
        pltpu.sync_copy(u_hbm.at[pl.ds(base, bpw)], idx_v)
        pltpu.sync_copy(tab_hbm.at[idx_v], rows_v)
        pltpu.sync_copy(rows_v, out_hbm.at[pl.ds(base, bpw)])

    return k(u, usr_table)


def _exp_scores(user_emb, relT_pad):
    B = user_emb.shape[0]

    def body(ue_ref, rt_ref, out_ref):
        out_ref[...] = jnp.exp(
            jnp.dot(ue_ref[...], rt_ref[...], preferred_element_type=jnp.float32)
        )

    return pl.pallas_call(
        body,
        out_shape=jax.ShapeDtypeStruct((B, NRELP), jnp.float32),
    )(user_emb, relT_pad)


def _gather_aggregate(v, adj, rel, ent_table, E):
    B = v.shape[0]
    bpw = B // NW

    @functools.partial(
        pl.kernel,
        out_type=(
            jax.ShapeDtypeStruct((B, DIM), jnp.float32),
            jax.ShapeDtypeStruct((B, NNB, DIM), jnp.float32),
            jax.ShapeDtypeStruct((B, NNB), jnp.float32),
        ),
        mesh=_mesh(),
        scratch_types=[
            pltpu.VMEM((bpw,), jnp.int32),
            pltpu.VMEM((bpw, NNB), jnp.int32),
            pltpu.VMEM((bpw * NNB,), jnp.int32),
            pltpu.VMEM((bpw, NNB), jnp.int32),
            pltpu.VMEM((bpw, DIM), jnp.float32),
            pltpu.VMEM((bpw, NRELP), jnp.float32),
            pltpu.VMEM((bpw * NNB, NNB), jnp.int32),
            pltpu.VMEM((bpw * NNB, NNB), jnp.int32),
            pltpu.VMEM((NNB * NNB,), jnp.int32),
            pltpu.VMEM((NNB * NNB, DIM), jnp.float32),
            pltpu.VMEM((NNB, DIM), jnp.float32),
            pltpu.VMEM((NNB, DIM), jnp.float32),
            pltpu.VMEM((bpw, DIM), jnp.float32),
            pltpu.VMEM((bpw, NNB), jnp.float32),
            pltpu.VMEM((L,), jnp.float32),
        ],
        compiler_params=pltpu.CompilerParams(
            use_tc_tiling_on_sc=False, needs_layout_passes=False),
    )
    def k(v_hbm, adj_hbm, rel_hbm, ent_hbm, e_hbm,
          sum0_hbm, sum1_hbm, w0_hbm,
          VL, E1, E1F, Q0, SV0, EC, E2F, Q1F, XIF, Xb, SV1b, SUM1b, SUM0, W0B,
          wbuf):
        wid = lax.axis_index("s") * NC + lax.axis_index("c")
        base = wid * bpw

        pltpu.sync_copy(v_hbm.at[pl.ds(base, bpw)], VL)
        pltpu.sync_copy(adj_hbm.at[VL], E1)
        pltpu.sync_copy(rel_hbm.at[VL], Q0)
        pltpu.sync_copy(ent_hbm.at[VL], SV0)
        pltpu.sync_copy(e_hbm.at[pl.ds(base, bpw)], EC)

        def flatten(i, carry):
            E1F[pl.ds(i * NNB, NNB)] = E1[i, :]
            return carry
        lax.fori_loop(0, bpw, flatten, 0)

        pltpu.sync_copy(adj_hbm.at[E1F], E2F)
        pltpu.sync_copy(rel_hbm.at[E1F], Q1F)

        def seg_weights(b_vec, q):
            e = plsc.load_gather(EC, [b_vec, q])
            s = jnp.sum(e)
            wbuf[...] = e
            return (jnp.zeros((L,), jnp.float32) + 1.0) / (
                jnp.zeros((L,), jnp.float32) + s)

        def body(b, carry):
            b_vec = jnp.zeros((L,), jnp.int32) + b
            for p in range(NNB):
                XIF[pl.ds(p * NNB, NNB)] = E2F[b * NNB + p, :]
            pltpu.sync_copy(ent_hbm.at[XIF], Xb)
            pltpu.sync_copy(ent_hbm.at[E1F.at[pl.ds(b * NNB, NNB)]], SV1b)
            for p in range(NNB):
                rs = seg_weights(b_vec, Q1F[b * NNB + p, :])
                acc0 = jnp.zeros((L,), jnp.float32)
                acc1 = jnp.zeros((L,), jnp.float32)
                for kk in range(NNB):
                    bk = plsc.load_gather(wbuf, [jnp.zeros((L,), jnp.int32) + kk])
                    acc0 = acc0 + bk * Xb[p * NNB + kk, 0:L]
                    acc1 = acc1 + bk * Xb[p * NNB + kk, L:DIM]
                SUM1b[p, 0:L] = acc0 * rs + SV1b[p, 0:L]
                SUM1b[p, L:DIM] = acc1 * rs + SV1b[p, L:DIM]
            pltpu.sync_copy(SUM1b, sum1_hbm.at[base + b])
            rs0 = seg_weights(b_vec, Q0[b, :])
            a0 = jnp.zeros((L,), jnp.float32)
            a1 = jnp.zeros((L,), jnp.float32)
            for kk in range(NNB):
                bk = plsc.load_gather(wbuf, [jnp.zeros((L,), jnp.int32) + kk])
                a0 = a0 + bk * SV1b[kk, 0:L]
                a1 = a1 + bk * SV1b[kk, L:DIM]
            W0B[b, :] = wbuf[...] * rs0
            SUM0[b, 0:L] = a0 * rs0 + SV0[b, 0:L]
            SUM0[b, L:DIM] = a1 * rs0 + SV0[b, L:DIM]
            return carry

        lax.fori_loop(0, bpw, body, 0)
        pltpu.sync_copy(SUM0, sum0_hbm.at[pl.ds(base, bpw)])
        pltpu.sync_copy(W0B, w0_hbm.at[pl.ds(base, bpw)])

    return k(v, adj, rel, ent_table, E)


def _dense_finish(user_emb, sum0, sum1_2d, w0, W0T, b0, W1T, b1):
    B = user_emb.shape[0]
    BB = 512
    grid = B // BB

    def body(ue_ref, s0_ref, s1_ref, w0_ref, w0t_ref, b0_ref, w1t_ref, b1_ref,
             out_ref):
        w0t = w0t_ref[...]
        b0v = b0_ref[...]
        w0w = w0_ref[...]
        aggtop = jnp.zeros((BB, DIM), jnp.float32)
        for kk in range(NNB):
            h1k = jax.nn.sigmoid(
                jnp.dot(s1_ref[:, kk * DIM:(kk + 1) * DIM], w0t,
                        preferred_element_type=jnp.float32) + b0v
            )
            aggtop = aggtop + w0w[:, kk:kk + 1] * h1k
        h0 = jax.nn.sigmoid(
            jnp.dot(s0_ref[...], w0t, preferred_element_type=jnp.float32) + b0v
        )
        item = jnp.tanh(
            jnp.dot(h0 + aggtop, w1t_ref[...], preferred_element_type=jnp.float32)
            + b1_ref[...]
        )
        out_ref[...] = jax.nn.sigmoid(jnp.sum(ue_ref[...] * item, axis=1))

    return pl.pallas_call(
        body,
        grid=(grid,),
        in_specs=[
            pl.BlockSpec((BB, DIM), lambda i: (i, 0)),
            pl.BlockSpec((BB, DIM), lambda i: (i, 0)),
            pl.BlockSpec((BB, NNB * DIM), lambda i: (i, 0)),
            pl.BlockSpec((BB, NNB), lambda i: (i, 0)),
            pl.BlockSpec((DIM, DIM), lambda i: (0, 0)),
            pl.BlockSpec((1, DIM), lambda i: (0, 0)),
            pl.BlockSpec((DIM, DIM), lambda i: (0, 0)),
            pl.BlockSpec((1, DIM), lambda i: (0, 0)),
        ],
        out_specs=pl.BlockSpec((BB,), lambda i: (i,)),
        out_shape=jax.ShapeDtypeStruct((B,), jnp.float32),
    )(user_emb, sum0, sum1_2d, w0, W0T, b0, W1T, b1)


def kernel(u, v, adj, rel, train_mode, usr_table, ent_table, rel_table,
           agg_W0, agg_b0, agg_W1, agg_b1):
    del train_mode
    B = v.shape[0]
    u = u.astype(jnp.int32)
    v = v.astype(jnp.int32)
    adj = adj.astype(jnp.int32)
    rel = rel.astype(jnp.int32)

    user_emb = _user_gather(u, usr_table)

    relT_pad = jnp.zeros((DIM, NRELP), jnp.float32).at[:, :rel_table.shape[0]].set(
        rel_table.T)
    E = _exp_scores(user_emb, relT_pad)

    sum0, sum1, w0 = _gather_aggregate(v, adj, rel, ent_table, E)

    return _dense_finish(
        user_emb, sum0, sum1.reshape(B, NNB * DIM), w0,
        agg_W0.T, agg_b0.reshape(1, DIM), agg_W1.T, agg_b1.reshape(1, DIM))


# speedup vs baseline: 17.7082x; 1.1502x over previous
import functools

import jax
import jax.numpy as jnp
from jax import lax
from jax.experimental import pallas as pl
from jax.experimental.pallas import tpu as pltpu
from jax.experimental.pallas import tpu_sc as plsc

DIM = 32
NNB = 16
NRELP = 48
NC, NS, L = 2, 16, 16
NW = NC * NS


def _mesh():
    return plsc.VectorSubcoreMesh(core_axis_name="c", subcore_axis_name="s")


def _user_gather(u, usr_table):
    B = u.shape[0]
    bpw = B // NW

    @functools.partial(
        pl.kernel,
        out_type=jax.ShapeDtypeStruct((B, DIM), jnp.float32),
        mesh=_mesh(),
        scratch_types=[
            pltpu.VMEM((bpw,), jnp.int32),
            pltpu.VMEM((bpw, DIM), jnp.float32),
        ],
        compiler_params=pltpu.CompilerParams(use_tc_tiling_on_sc=False),
    )
    def k(u_hbm, tab_hbm, out_hbm, idx_v, rows_v):
        wid = lax.axis_index("s") * NC + lax.axis_index("c")
        base = wid * bpw
        pltpu.sync_copy(u_hbm.at[pl.ds(base, bpw)], idx_v)
        pltpu.sync_copy(tab_hbm.at[idx_v], rows_v)
        pltpu.sync_copy(rows_v, out_hbm.at[pl.ds(base, bpw)])

    return k(u, usr_table)


def _exp_scores(user_emb, relT_pad):
    B = user_emb.shape[0]

    def body(ue_ref, rt_ref, out_ref):
        out_ref[...] = jnp.exp(
            jnp.dot(ue_ref[...], rt_ref[...], preferred_element_type=jnp.float32)
        )

    return pl.pallas_call(
        body,
        out_shape=jax.ShapeDtypeStruct((B, NRELP), jnp.float32),
    )(user_emb, relT_pad)


def _gather_aggregate(v, adj, rel, ent_table, E):
    B = v.shape[0]
    bpw = B // NW

    @functools.partial(
        pl.kernel,
        out_type=(
            jax.ShapeDtypeStruct((B, DIM), jnp.float32),
            jax.ShapeDtypeStruct((B, NNB, DIM), jnp.float32),
            jax.ShapeDtypeStruct((B, NNB), jnp.float32),
        ),
        mesh=_mesh(),
        scratch_types=[
            pltpu.VMEM((bpw,), jnp.int32),
            pltpu.VMEM((bpw, NNB), jnp.int32),
            pltpu.VMEM((bpw * NNB,), jnp.int32),
            pltpu.VMEM((bpw, NNB), jnp.int32),
            pltpu.VMEM((bpw, DIM), jnp.float32),
            pltpu.VMEM((bpw, NRELP), jnp.float32),
            pltpu.VMEM((bpw * NNB, NNB), jnp.int32),
            pltpu.VMEM((bpw * NNB, NNB), jnp.int32),
            pltpu.VMEM((2, NNB * NNB), jnp.int32),
            pltpu.VMEM((2, NNB * NNB, DIM), jnp.float32),
            pltpu.VMEM((2, NNB, DIM), jnp.float32),
            pltpu.VMEM((2, NNB, DIM), jnp.float32),
            pltpu.VMEM((bpw, DIM), jnp.float32),
            pltpu.VMEM((bpw, NNB), jnp.float32),
            pltpu.VMEM((L,), jnp.float32),
            pltpu.SemaphoreType.DMA,
            pltpu.SemaphoreType.DMA,
            pltpu.SemaphoreType.DMA,
            pltpu.SemaphoreType.DMA,
            pltpu.SemaphoreType.DMA,
            pltpu.SemaphoreType.DMA,
        ],
        compiler_params=pltpu.CompilerParams(
            use_tc_tiling_on_sc=False, needs_layout_passes=False),
    )
    def k(v_hbm, adj_hbm, rel_hbm, ent_hbm, e_hbm,
          sum0_hbm, sum1_hbm, w0_hbm,
          VL, E1, E1F, Q0, SV0, EC, E2F, Q1F, XIF, Xb, SV1b, SUM1b, SUM0, W0B,
          wbuf, semX0, semX1, semS0, semS1, semW0, semW1):
        semX = (semX0, semX1)
        semS = (semS0, semS1)
        semW = (semW0, semW1)
        wid = lax.axis_index("s") * NC + lax.axis_index("c")
        base = wid * bpw

        pltpu.sync_copy(v_hbm.at[pl.ds(base, bpw)], VL)
        pltpu.sync_copy(adj_hbm.at[VL], E1)
        pltpu.sync_copy(rel_hbm.at[VL], Q0)
        pltpu.sync_copy(ent_hbm.at[VL], SV0)
        pltpu.sync_copy(e_hbm.at[pl.ds(base, bpw)], EC)

        def flatten(i, carry):
            E1F[pl.ds(i * NNB, NNB)] = E1[i, :]
            return carry
        lax.fori_loop(0, bpw, flatten, 0)

        pltpu.sync_copy(adj_hbm.at[E1F], E2F)
        pltpu.sync_copy(rel_hbm.at[E1F], Q1F)

        def seg_weights(b_vec, q):
            e = plsc.load_gather(EC, [b_vec, q])
            s = jnp.sum(e)
            wbuf[...] = e
            return (jnp.zeros((L,), jnp.float32) + 1.0) / (
                jnp.zeros((L,), jnp.float32) + s)

        def fire(b, j):
            for p in range(NNB):
                XIF[j, pl.ds(p * NNB, NNB)] = E2F[b * NNB + p, :]
            pltpu.async_copy(ent_hbm.at[XIF.at[j]], Xb.at[j], semX[j])
            pltpu.async_copy(ent_hbm.at[E1F.at[pl.ds(b * NNB, NNB)]],
                             SV1b.at[j], semS[j])

        fire(0, 0)
        fire(1, 1)

        def outer(i, carry):
            for j in range(2):
                b = i * 2 + j
                pltpu.make_async_copy(
                    ent_hbm.at[XIF.at[j]], Xb.at[j], semX[j]).wait()
                pltpu.make_async_copy(
                    ent_hbm.at[E1F.at[pl.ds(b * NNB, NNB)]],
                    SV1b.at[j], semS[j]).wait()

                @pl.when(b >= 2)
                def _():
                    pltpu.make_async_copy(
                        SUM1b.at[j], sum1_hbm.at[base + b - 2], semW[j]).wait()

                b_vec = jnp.zeros((L,), jnp.int32) + b
                for p in range(NNB):
                    rs = seg_weights(b_vec, Q1F[b * NNB + p, :])
                    acc0 = jnp.zeros((L,), jnp.float32)
                    acc1 = jnp.zeros((L,), jnp.float32)
                    for kk in range(NNB):
                        bk = plsc.load_gather(
                            wbuf, [jnp.zeros((L,), jnp.int32) + kk])
                        acc0 = acc0 + bk * Xb[j, p * NNB + kk, 0:L]
                        acc1 = acc1 + bk * Xb[j, p * NNB + kk, L:DIM]
                    SUM1b[j, p, 0:L] = acc0 * rs + SV1b[j, p, 0:L]
                    SUM1b[j, p, L:DIM] = acc1 * rs + SV1b[j, p, L:DIM]
                pltpu.async_copy(SUM1b.at[j], sum1_hbm.at[base + b], semW[j])
                rs0 = seg_weights(b_vec, Q0[b, :])
                a0 = jnp.zeros((L,), jnp.float32)
                a1 = jnp.zeros((L,), jnp.float32)
                for kk in range(NNB):
                    bk = plsc.load_gather(
                        wbuf, [jnp.zeros((L,), jnp.int32) + kk])
                    a0 = a0 + bk * SV1b[j, kk, 0:L]
                    a1 = a1 + bk * SV1b[j, kk, L:DIM]
                W0B[b, :] = wbuf[...] * rs0
                SUM0[b, 0:L] = a0 * rs0 + SV0[b, 0:L]
                SUM0[b, L:DIM] = a1 * rs0 + SV0[b, L:DIM]

                @pl.when(b + 2 < bpw)
                def _():
                    fire(b + 2, j)
            return carry

        lax.fori_loop(0, bpw // 2, outer, 0)
        pltpu.make_async_copy(
            SUM1b.at[0], sum1_hbm.at[base + bpw - 2], semW[0]).wait()
        pltpu.make_async_copy(
            SUM1b.at[1], sum1_hbm.at[base + bpw - 1], semW[1]).wait()
        pltpu.sync_copy(SUM0, sum0_hbm.at[pl.ds(base, bpw)])
        pltpu.sync_copy(W0B, w0_hbm.at[pl.ds(base, bpw)])

    return k(v, adj, rel, ent_table, E)


def _dense_finish(user_emb, sum0, sum1_2d, w0, W0T, b0, W1T, b1):
    B = user_emb.shape[0]
    BB = 512
    grid = B // BB

    def body(ue_ref, s0_ref, s1_ref, w0_ref, w0t_ref, b0_ref, w1t_ref, b1_ref,
             out_ref):
        w0t = w0t_ref[...]
        b0v = b0_ref[...]
        w0w = w0_ref[...]
        aggtop = jnp.zeros((BB, DIM), jnp.float32)
        for kk in range(NNB):
            h1k = jax.nn.sigmoid(
                jnp.dot(s1_ref[:, kk * DIM:(kk + 1) * DIM], w0t,
                        preferred_element_type=jnp.float32) + b0v
            )
            aggtop = aggtop + w0w[:, kk:kk + 1] * h1k
        h0 = jax.nn.sigmoid(
            jnp.dot(s0_ref[...], w0t, preferred_element_type=jnp.float32) + b0v
        )
        item = jnp.tanh(
            jnp.dot(h0 + aggtop, w1t_ref[...], preferred_element_type=jnp.float32)
            + b1_ref[...]
        )
        out_ref[...] = jax.nn.sigmoid(jnp.sum(ue_ref[...] * item, axis=1))

    return pl.pallas_call(
        body,
        grid=(grid,),
        in_specs=[
            pl.BlockSpec((BB, DIM), lambda i: (i, 0)),
            pl.BlockSpec((BB, DIM), lambda i: (i, 0)),
            pl.BlockSpec((BB, NNB * DIM), lambda i: (i, 0)),
            pl.BlockSpec((BB, NNB), lambda i: (i, 0)),
            pl.BlockSpec((DIM, DIM), lambda i: (0, 0)),
            pl.BlockSpec((1, DIM), lambda i: (0, 0)),
            pl.BlockSpec((DIM, DIM), lambda i: (0, 0)),
            pl.BlockSpec((1, DIM), lambda i: (0, 0)),
        ],
        out_specs=pl.BlockSpec((BB,), lambda i: (i,)),
        out_shape=jax.ShapeDtypeStruct((B,), jnp.float32),
    )(user_emb, sum0, sum1_2d, w0, W0T, b0, W1T, b1)


def kernel(u, v, adj, rel, train_mode, usr_table, ent_table, rel_table,
           agg_W0, agg_b0, agg_W1, agg_b1):
    del train_mode
    B = v.shape[0]
    u = u.astype(jnp.int32)
    v = v.astype(jnp.int32)
    adj = adj.astype(jnp.int32)
    rel = rel.astype(jnp.int32)

    user_emb = _user_gather(u, usr_table)

    relT_pad = jnp.zeros((DIM, NRELP), jnp.float32).at[:, :rel_table.shape[0]].set(
        rel_table.T)
    E = _exp_scores(user_emb, relT_pad)

    sum0, sum1, w0 = _gather_aggregate(v, adj, rel, ent_table, E)

    return _dense_finish(
        user_emb, sum0, sum1.reshape(B, NNB * DIM), w0,
        agg_W0.T, agg_b0.reshape(1, DIM), agg_W1.T, agg_b1.reshape(1, DIM))


# speedup vs baseline: 19.6728x; 1.1109x over previous
import functools

import jax
import jax.numpy as jnp
from jax import lax
from jax.experimental import pallas as pl
from jax.experimental.pallas import tpu as pltpu
from jax.experimental.pallas import tpu_sc as plsc

DIM = 32
NNB = 16
NRELP = 48
NC, NS, L = 2, 16, 16
NW = NC * NS


def _mesh():
    return plsc.VectorSubcoreMesh(core_axis_name="c", subcore_axis_name="s")


def _gather_aggregate(u, v, adj, rel, usr_table, ent_table, rtT):
    B = v.shape[0]
    bpw = B // NW

    @functools.partial(
        pl.kernel,
        out_type=(
            jax.ShapeDtypeStruct((B, DIM), jnp.float32),
            jax.ShapeDtypeStruct((B, DIM), jnp.float32),
            jax.ShapeDtypeStruct((B, NNB * DIM), jnp.float32),
            jax.ShapeDtypeStruct((B, NNB), jnp.float32),
        ),
        mesh=_mesh(),
        scratch_types=[
            pltpu.VMEM((bpw,), jnp.int32),
            pltpu.VMEM((bpw, DIM), jnp.float32),
            pltpu.VMEM((DIM, NRELP), jnp.float32),
            pltpu.VMEM((NRELP,), jnp.float32),
            pltpu.VMEM((bpw,), jnp.int32),
            pltpu.VMEM((bpw, NNB), jnp.int32),
            pltpu.VMEM((bpw * NNB,), jnp.int32),
            pltpu.VMEM((bpw, NNB), jnp.int32),
            pltpu.VMEM((bpw, DIM), jnp.float32),
            pltpu.VMEM((bpw * NNB, NNB), jnp.int32),
            pltpu.VMEM((bpw * NNB, NNB), jnp.int32),
            pltpu.VMEM((2, NNB * NNB), jnp.int32),
            pltpu.VMEM((2, NNB * NNB, DIM), jnp.float32),
            pltpu.VMEM((2, NNB, DIM), jnp.float32),
            pltpu.VMEM((2, NNB * DIM), jnp.float32),
            pltpu.VMEM((bpw, DIM), jnp.float32),
            pltpu.VMEM((bpw, NNB), jnp.float32),
            pltpu.VMEM((L,), jnp.float32),
            pltpu.SemaphoreType.DMA,
            pltpu.SemaphoreType.DMA,
            pltpu.SemaphoreType.DMA,
            pltpu.SemaphoreType.DMA,
            pltpu.SemaphoreType.DMA,
            pltpu.SemaphoreType.DMA,
        ],
        compiler_params=pltpu.CompilerParams(
            use_tc_tiling_on_sc=False, needs_layout_passes=False),
    )
    def k(u_hbm, v_hbm, adj_hbm, rel_hbm, usr_hbm, ent_hbm, rtT_hbm,
          ue_hbm, sum0_hbm, sum1_hbm, w0_hbm,
          UL, UE, RT, ECb, VL, E1, E1F, Q0, SV0, E2F, Q1F, XIF, Xb, SV1b,
          SUM1b, SUM0, W0B, wbuf,
          semX0, semX1, semS0, semS1, semW0, semW1):
        semX = (semX0, semX1)
        semS = (semS0, semS1)
        semW = (semW0, semW1)
        wid = lax.axis_index("s") * NC + lax.axis_index("c")
        base = wid * bpw

        pltpu.sync_copy(u_hbm.at[pl.ds(base, bpw)], UL)
        pltpu.sync_copy(v_hbm.at[pl.ds(base, bpw)], VL)
        pltpu.sync_copy(rtT_hbm, RT)
        pltpu.sync_copy(usr_hbm.at[UL], UE)
        pltpu.sync_copy(adj_hbm.at[VL], E1)
        pltpu.sync_copy(rel_hbm.at[VL], Q0)
        pltpu.sync_copy(ent_hbm.at[VL], SV0)
        pltpu.sync_copy(UE, ue_hbm.at[pl.ds(base, bpw)])

        def flatten(i, carry):
            E1F[pl.ds(i * NNB, NNB)] = E1[i, :]
            return carry
        lax.fori_loop(0, bpw, flatten, 0)

        pltpu.sync_copy(adj_hbm.at[E1F], E2F)
        pltpu.sync_copy(rel_hbm.at[E1F], Q1F)

        def exp_scores(b_vec):
            u0 = jnp.zeros((L,), jnp.float32)
            u1 = jnp.zeros((L,), jnp.float32)
            u2 = jnp.zeros((L,), jnp.float32)
            for d in range(DIM):
                ud = plsc.load_gather(
                    UE, [b_vec, jnp.zeros((L,), jnp.int32) + d])
                u0 = u0 + ud * RT[d, 0:L]
                u1 = u1 + ud * RT[d, L:2 * L]
                u2 = u2 + ud * RT[d, 2 * L:3 * L]
            ECb[pl.ds(0, L)] = jnp.exp(u0)
            ECb[pl.ds(L, L)] = jnp.exp(u1)
            ECb[pl.ds(2 * L, L)] = jnp.exp(u2)

        def seg_weights(q):
            e = plsc.load_gather(ECb, [q])
            s = jnp.sum(e)
            wbuf[...] = e
            return (jnp.zeros((L,), jnp.float32) + 1.0) / (
                jnp.zeros((L,), jnp.float32) + s)

        def fire(b, j):
            for p in range(NNB):
                XIF[j, pl.ds(p * NNB, NNB)] = E2F[b * NNB + p, :]
            pltpu.async_copy(ent_hbm.at[XIF.at[j]], Xb.at[j], semX[j])
            pltpu.async_copy(ent_hbm.at[E1F.at[pl.ds(b * NNB, NNB)]],
                             SV1b.at[j], semS[j])

        fire(0, 0)
        fire(1, 1)

        def outer(i, carry):
            for j in range(2):
                b = i * 2 + j
                b_vec = jnp.zeros((L,), jnp.int32) + b
                exp_scores(b_vec)
                pltpu.make_async_copy(
                    ent_hbm.at[XIF.at[j]], Xb.at[j], semX[j]).wait()
                pltpu.make_async_copy(
                    ent_hbm.at[E1F.at[pl.ds(b * NNB, NNB)]],
                    SV1b.at[j], semS[j]).wait()

                @pl.when(b >= 2)
                def _():
                    pltpu.make_async_copy(
                        SUM1b.at[j], sum1_hbm.at[base + b - 2], semW[j]).wait()

                for p in range(NNB):
                    rs = seg_weights(Q1F[b * NNB + p, :])
                    acc0 = jnp.zeros((L,), jnp.float32)
                    acc1 = jnp.zeros((L,), jnp.float32)
                    for kk in range(NNB):
                        bk = plsc.load_gather(
                            wbuf, [jnp.zeros((L,), jnp.int32) + kk])
                        acc0 = acc0 + bk * Xb[j, p * NNB + kk, 0:L]
                        acc1 = acc1 + bk * Xb[j, p * NNB + kk, L:DIM]
                    SUM1b[j, pl.ds(p * DIM, L)] = acc0 * rs + SV1b[j, p, 0:L]
                    SUM1b[j, pl.ds(p * DIM + L, L)] = (
                        acc1 * rs + SV1b[j, p, L:DIM])
                pltpu.async_copy(SUM1b.at[j], sum1_hbm.at[base + b], semW[j])
                rs0 = seg_weights(Q0[b, :])
                a0 = jnp.zeros((L,), jnp.float32)
                a1 = jnp.zeros((L,), jnp.float32)
                for kk in range(NNB):
                    bk = plsc.load_gather(
                        wbuf, [jnp.zeros((L,), jnp.int32) + kk])
                    a0 = a0 + bk * SV1b[j, kk, 0:L]
                    a1 = a1 + bk * SV1b[j, kk, L:DIM]
                W0B[b, :] = wbuf[...] * rs0
                SUM0[b, 0:L] = a0 * rs0 + SV0[b, 0:L]
                SUM0[b, L:DIM] = a1 * rs0 + SV0[b, L:DIM]

                @pl.when(b + 2 < bpw)
                def _():
                    fire(b + 2, j)
            return carry

        lax.fori_loop(0, bpw // 2, outer, 0)
        pltpu.make_async_copy(
            SUM1b.at[0], sum1_hbm.at[base + bpw - 2], semW[0]).wait()
        pltpu.make_async_copy(
            SUM1b.at[1], sum1_hbm.at[base + bpw - 1], semW[1]).wait()
        pltpu.sync_copy(SUM0, sum0_hbm.at[pl.ds(base, bpw)])
        pltpu.sync_copy(W0B, w0_hbm.at[pl.ds(base, bpw)])

    return k(u, v, adj, rel, usr_table, ent_table, rtT)


def _dense_finish(user_emb, sum0, sum1_2d, w0, W0T, b0, W1T, b1):
    B = user_emb.shape[0]
    BB = 512
    grid = B // BB

    def body(ue_ref, s0_ref, s1_ref, w0_ref, w0t_ref, b0_ref, w1t_ref, b1_ref,
             out_ref):
        w0t = w0t_ref[...]
        b0v = b0_ref[...]
        w0w = w0_ref[...]
        aggtop = jnp.zeros((BB, DIM), jnp.float32)
        for kk in range(NNB):
            h1k = jax.nn.sigmoid(
                jnp.dot(s1_ref[:, kk * DIM:(kk + 1) * DIM], w0t,
                        preferred_element_type=jnp.float32) + b0v
            )
            aggtop = aggtop + w0w[:, kk:kk + 1] * h1k
        h0 = jax.nn.sigmoid(
            jnp.dot(s0_ref[...], w0t, preferred_element_type=jnp.float32) + b0v
        )
        item = jnp.tanh(
            jnp.dot(h0 + aggtop, w1t_ref[...], preferred_element_type=jnp.float32)
            + b1_ref[...]
        )
        out_ref[...] = jax.nn.sigmoid(jnp.sum(ue_ref[...] * item, axis=1))

    return pl.pallas_call(
        body,
        grid=(grid,),
        in_specs=[
            pl.BlockSpec((BB, DIM), lambda i: (i, 0)),
            pl.BlockSpec((BB, DIM), lambda i: (i, 0)),
            pl.BlockSpec((BB, NNB * DIM), lambda i: (i, 0)),
            pl.BlockSpec((BB, NNB), lambda i: (i, 0)),
            pl.BlockSpec((DIM, DIM), lambda i: (0, 0)),
            pl.BlockSpec((1, DIM), lambda i: (0, 0)),
            pl.BlockSpec((DIM, DIM), lambda i: (0, 0)),
            pl.BlockSpec((1, DIM), lambda i: (0, 0)),
        ],
        out_specs=pl.BlockSpec((BB,), lambda i: (i,)),
        out_shape=jax.ShapeDtypeStruct((B,), jnp.float32),
    )(user_emb, sum0, sum1_2d, w0, W0T, b0, W1T, b1)


def kernel(u, v, adj, rel, train_mode, usr_table, ent_table, rel_table,
           agg_W0, agg_b0, agg_W1, agg_b1):
    del train_mode
    u = u.astype(jnp.int32)
    v = v.astype(jnp.int32)
    adj = adj.astype(jnp.int32)
    rel = rel.astype(jnp.int32)

    rtT = jnp.zeros((DIM, NRELP), jnp.float32).at[:, :rel_table.shape[0]].set(
        rel_table.T)

    user_emb, sum0, sum1, w0 = _gather_aggregate(
        u, v, adj, rel, usr_table, ent_table, rtT)

    return _dense_finish(
        user_emb, sum0, sum1, w0,
        agg_W0.T, agg_b0.reshape(1, DIM), agg_W1.T, agg_b1.reshape(1, DIM))
